# baseline (speedup 1.0000x reference)
import functools

import jax
import jax.numpy as jnp
from jax import lax
from jax.experimental import pallas as pl
from jax.experimental.pallas import tpu as pltpu
from jax.experimental.pallas import tpu_sc as plsc

_D = 128
_C = 16
_V = 4096
_NVAR = 10000
_NGENE = 20000
_L = 16
_NW = 32
_D_PER_W = _D // _NW
_VH = _V // 2


def _sc_gather(genotypes, sel, baseline_log, g2g):
    mesh = plsc.VectorSubcoreMesh(core_axis_name="core", subcore_axis_name="sub")

    @functools.partial(
        pl.kernel,
        out_type=(
            jax.ShapeDtypeStruct((_D, _V), jnp.float32),
            jax.ShapeDtypeStruct((_C, _V), jnp.float32),
        ),
        mesh=mesh,
        compiler_params=pltpu.CompilerParams(needs_layout_passes=False),
        scratch_types=[
            pltpu.VMEM((_NVAR,), jnp.float32),
            pltpu.VMEM((_NVAR,), jnp.float32),
            pltpu.VMEM((_NGENE,), jnp.float32),
            pltpu.VMEM((_V,), jnp.int32),
            pltpu.VMEM((_VH,), jnp.int32),
            pltpu.VMEM((_V,), jnp.float32),
            pltpu.VMEM((_V,), jnp.float32),
            pltpu.VMEM((_VH,), jnp.float32),
            pltpu.SemaphoreType.DMA,
            pltpu.SemaphoreType.DMA,
            pltpu.SemaphoreType.DMA,
            pltpu.SemaphoreType.DMA,
            pltpu.SemaphoreType.DMA,
            pltpu.SemaphoreType.DMA,
            pltpu.SemaphoreType.DMA,
        ],
    )
    def body(gen_hbm, sel_hbm, base_hbm, g2g_hbm, gg_hbm, bg_hbm,
             grow0, grow1, brow_v, sel_v, g2g_v, gout0, gout1, bout_v,
             sem_sel, sem_row0, sem_row1, sem_bl, sem_g2g, sem_go0, sem_go1):
        wid = lax.axis_index("sub") * 2 + lax.axis_index("core")
        d0 = wid * _D_PER_W
        c = wid % _C
        half = wid // _C

        cp_sel = pltpu.async_copy(sel_hbm, sel_v, sem_sel)
        cp_bl = pltpu.async_copy(base_hbm.at[c], brow_v, sem_bl)
        cp_g2g = pltpu.async_copy(g2g_hbm.at[pl.ds(half * _VH, _VH)], g2g_v, sem_g2g)

        rows = (grow0, grow1)
        row_sems = (sem_row0, sem_row1)
        gouts = (gout0, gout1)
        go_sems = (sem_go0, sem_go1)

        cur = pltpu.async_copy(gen_hbm.at[d0], grow0, sem_row0)
        cp_sel.wait()
        out_cps = [None, None]
        for j in range(_D_PER_W):
            if j + 1 < _D_PER_W:
                nxt = pltpu.async_copy(gen_hbm.at[d0 + j + 1],
                                       rows[(j + 1) % 2], row_sems[(j + 1) % 2])
            cur.wait()
            row = rows[j % 2]
            gout = gouts[j % 2]
            if out_cps[j % 2] is not None:
                out_cps[j % 2].wait()

            @plsc.parallel_loop(0, _V // _L, unroll=8)
            def gbody(i):
                off = pl.multiple_of(i * _L, _L)
                idx = sel_v[pl.ds(off, _L)]
                gout[pl.ds(off, _L)] = plsc.load_gather(row, [idx])

            out_cps[j % 2] = pltpu.async_copy(gout, gg_hbm.at[d0 + j], go_sems[j % 2])
            if j + 1 < _D_PER_W:
                cur = nxt

        cp_g2g.wait()
        cp_bl.wait()

        @plsc.parallel_loop(0, _VH // _L, unroll=8)
        def bbody(i):
            off = pl.multiple_of(i * _L, _L)
            idx = g2g_v[pl.ds(off, _L)]
            bout_v[pl.ds(off, _L)] = plsc.load_gather(brow_v, [idx])

        pltpu.sync_copy(bout_v, bg_hbm.at[c, pl.ds(half * _VH, _VH)])
        for cp in out_cps:
            if cp is not None:
                cp.wait()

    return body(genotypes, sel, baseline_log, g2g)


_DBLK = 64


def _tc_dense_body(gg_ref, bg_hbm, fc_hbm, lib_ref, out_ref, bg_v, fc_v, sem):
    @pl.when(pl.program_id(0) == 0)
    def _():
        pltpu.async_copy(bg_hbm, bg_v, sem).wait()
        pltpu.async_copy(fc_hbm, fc_v, sem).wait()

    g = gg_ref[:][:, None, :]
    loglib = jnp.log(lib_ref[:])
    e = bg_v[:][None, :, :] + g * fc_v[:][None, :, :] + loglib[:, :, None]
    out_ref[:] = jnp.exp(e)


def _tc_dense(gg, bg, fc, lib):
    return pl.pallas_call(
        _tc_dense_body,
        grid=(_D // _DBLK,),
        in_specs=[
            pl.BlockSpec((_DBLK, _V), lambda i: (i, 0)),
            pl.BlockSpec(memory_space=pl.ANY),
            pl.BlockSpec(memory_space=pl.ANY),
            pl.BlockSpec((_DBLK, _C), lambda i: (i, 0)),
        ],
        out_specs=pl.BlockSpec((_DBLK, _C, _V), lambda i: (i, 0, 0)),
        out_shape=jax.ShapeDtypeStruct((_D, _C, _V), jnp.float32),
        scratch_shapes=[
            pltpu.VMEM((_C, _V), jnp.float32),
            pltpu.VMEM((_C, _V), jnp.float32),
            pltpu.SemaphoreType.DMA,
        ],
    )(gg, bg, fc, lib)


def kernel(fc_log, genotypes, expression_obs, variantxgene_to_gene,
           local_variant_to_local_variantxgene_selector, variantxgene_to_local_gene,
           lib, baseline_log, dispersion_log):
    gg, bg = _sc_gather(genotypes, local_variant_to_local_variantxgene_selector,
                        baseline_log, variantxgene_to_gene)
    return _tc_dense(gg, bg, fc_log, lib)


# speedup vs baseline: 1.0340x; 1.0340x over previous
import functools

import jax
import jax.numpy as jnp
from jax import lax
from jax.experimental import pallas as pl
from jax.experimental.pallas import tpu as pltpu
from jax.experimental.pallas import tpu_sc as plsc

_D = 128
_C = 16
_V = 4096
_NVAR = 10000
_NGENE = 20000
_L = 16
_NW = 32
_D_PER_W = _D // _NW
_VH = _V // 2


def _sc_gather(genotypes, sel, baseline_log, g2g):
    mesh = plsc.VectorSubcoreMesh(core_axis_name="core", subcore_axis_name="sub")

    @functools.partial(
        pl.kernel,
        out_type=(
            jax.ShapeDtypeStruct((_D, _V), jnp.float32),
            jax.ShapeDtypeStruct((_C, _V), jnp.float32),
        ),
        mesh=mesh,
        compiler_params=pltpu.CompilerParams(needs_layout_passes=False),
        scratch_types=[
            pltpu.VMEM((_NVAR,), jnp.float32),
            pltpu.VMEM((_NVAR,), jnp.float32),
            pltpu.VMEM((_NGENE,), jnp.float32),
            pltpu.VMEM((_V,), jnp.int32),
            pltpu.VMEM((_VH,), jnp.int32),
            pltpu.VMEM((_V,), jnp.float32),
            pltpu.VMEM((_V,), jnp.float32),
            pltpu.VMEM((_VH,), jnp.float32),
            pltpu.SemaphoreType.DMA,
            pltpu.SemaphoreType.DMA,
            pltpu.SemaphoreType.DMA,
            pltpu.SemaphoreType.DMA,
            pltpu.SemaphoreType.DMA,
            pltpu.SemaphoreType.DMA,
            pltpu.SemaphoreType.DMA,
        ],
    )
    def body(gen_hbm, sel_hbm, base_hbm, g2g_hbm, gg_hbm, bg_hbm,
             grow0, grow1, brow_v, sel_v, g2g_v, gout0, gout1, bout_v,
             sem_sel, sem_row0, sem_row1, sem_bl, sem_g2g, sem_go0, sem_go1):
        wid = lax.axis_index("sub") * 2 + lax.axis_index("core")
        d0 = wid * _D_PER_W
        c = wid % _C
        half = wid // _C

        cp_sel = pltpu.async_copy(sel_hbm, sel_v, sem_sel)
        cp_bl = pltpu.async_copy(base_hbm.at[c], brow_v, sem_bl)
        cp_g2g = pltpu.async_copy(g2g_hbm.at[pl.ds(half * _VH, _VH)], g2g_v, sem_g2g)

        rows = (grow0, grow1)
        row_sems = (sem_row0, sem_row1)
        gouts = (gout0, gout1)
        go_sems = (sem_go0, sem_go1)

        cur = pltpu.async_copy(gen_hbm.at[d0], grow0, sem_row0)
        cp_sel.wait()
        out_cps = [None, None]
        for j in range(_D_PER_W):
            if j + 1 < _D_PER_W:
                nxt = pltpu.async_copy(gen_hbm.at[d0 + j + 1],
                                       rows[(j + 1) % 2], row_sems[(j + 1) % 2])
            cur.wait()
            row = rows[j % 2]
            gout = gouts[j % 2]
            if out_cps[j % 2] is not None:
                out_cps[j % 2].wait()

            @plsc.parallel_loop(0, _V // _L, unroll=8)
            def gbody(i):
                off = pl.multiple_of(i * _L, _L)
                idx = sel_v[pl.ds(off, _L)]
                gout[pl.ds(off, _L)] = plsc.load_gather(row, [idx])

            out_cps[j % 2] = pltpu.async_copy(gout, gg_hbm.at[d0 + j], go_sems[j % 2])
            if j + 1 < _D_PER_W:
                cur = nxt

        cp_g2g.wait()
        cp_bl.wait()

        @plsc.parallel_loop(0, _VH // _L, unroll=8)
        def bbody(i):
            off = pl.multiple_of(i * _L, _L)
            idx = g2g_v[pl.ds(off, _L)]
            bout_v[pl.ds(off, _L)] = plsc.load_gather(brow_v, [idx])

        pltpu.sync_copy(bout_v, bg_hbm.at[c, pl.ds(half * _VH, _VH)])
        for cp in out_cps:
            if cp is not None:
                cp.wait()

    return body(genotypes, sel, baseline_log, g2g)


_DBLK = 32


def _tc_dense_body(gg_ref, bg_hbm, fc_hbm, lib_ref, out_ref, bg_v, fc_v, sem):
    @pl.when(pl.program_id(0) == 0)
    def _():
        pltpu.async_copy(bg_hbm, bg_v, sem).wait()
        pltpu.async_copy(fc_hbm, fc_v, sem).wait()

    g = gg_ref[:][:, None, :]
    loglib = jnp.log(lib_ref[:])
    e = bg_v[:][None, :, :] + g * fc_v[:][None, :, :] + loglib[:, :, None]
    out_ref[:] = jnp.exp(e)


def _tc_dense(gg, bg, fc, lib):
    return pl.pallas_call(
        _tc_dense_body,
        grid=(_D // _DBLK,),
        in_specs=[
            pl.BlockSpec((_DBLK, _V), lambda i: (i, 0)),
            pl.BlockSpec(memory_space=pl.ANY),
            pl.BlockSpec(memory_space=pl.ANY),
            pl.BlockSpec((_DBLK, _C), lambda i: (i, 0)),
        ],
        out_specs=pl.BlockSpec((_DBLK, _C, _V), lambda i: (i, 0, 0)),
        out_shape=jax.ShapeDtypeStruct((_D, _C, _V), jnp.float32),
        scratch_shapes=[
            pltpu.VMEM((_C, _V), jnp.float32),
            pltpu.VMEM((_C, _V), jnp.float32),
            pltpu.SemaphoreType.DMA,
        ],
    )(gg, bg, fc, lib)


def kernel(fc_log, genotypes, expression_obs, variantxgene_to_gene,
           local_variant_to_local_variantxgene_selector, variantxgene_to_local_gene,
           lib, baseline_log, dispersion_log):
    gg, bg = _sc_gather(genotypes, local_variant_to_local_variantxgene_selector,
                        baseline_log, variantxgene_to_gene)
    return _tc_dense(gg, bg, fc_log, lib)
